# baseline (speedup 1.0000x reference)
import jax
import jax.numpy as jnp
from jax import lax
from jax.experimental import pallas as pl
from jax.experimental.pallas import tpu as pltpu

N_CLS = 128
N_CTX = 32
HALF = N_CTX // 2
N_EXPERTS = 64
TOP_K = 4
CTX_DIM = 768
SEQ_LEN = 77
SUF_LEN = SEQ_LEN - 1 - N_CTX
LANES = 16
BLK = 8


def _fused_body(rad_ref, w_gate_ref, shared_ref, ws_w_ref, ws_b_ref,
                ctxg_ref, ctxc_ref, prefix_ref, suffix_ref,
                out_ref, aux_ref, mid_ref):
    i = pl.program_id(0)

    @pl.when(i == 0)
    def _():
        ctx_s = lax.dot_general(shared_ref[...], ws_w_ref[...],
                                (((1,), (1,)), ((), ())),
                                preferred_element_type=jnp.float32)
        logits = lax.dot_general(rad_ref[...], w_gate_ref[...],
                                 (((1,), (0,)), ((), ())),
                                 preferred_element_type=jnp.float32)
        iota = lax.broadcasted_iota(jnp.int32, (1, N_EXPERTS), 1)
        v = logits
        vals, idxs = [], []
        for _ in range(TOP_K):
            s = jnp.max(v)
            e = jnp.min(jnp.where(v == s, iota, N_EXPERTS))
            vals.append(s)
            idxs.append(e)
            v = jnp.where(iota == e, -jnp.inf, v)
        m = vals[0]
        exps = [jnp.exp(val - m) for val in vals]
        tot = exps[0] + exps[1] + exps[2] + exps[3]
        gs = [ex / tot for ex in exps]

        g64 = jnp.zeros((1, N_EXPERTS), jnp.float32)
        for k in range(TOP_K):
            g64 = jnp.where(iota == idxs[k], gs[k], g64)
        s1 = jnp.sum(g64)
        s2 = jnp.sum(g64 * g64)
        mean = s1 / N_EXPERTS
        var = (s2 - N_EXPERTS * mean * mean) / (N_EXPERTS - 1)
        aux_ref[...] = jnp.full((1, 1), var / (mean * mean + 1e-10),
                                jnp.float32)

        rr = lax.broadcasted_iota(jnp.int32, (HALF - 1, N_EXPERTS * (HALF - 1)), 0)
        cc = lax.broadcasted_iota(jnp.int32, (HALF - 1, N_EXPERTS * (HALF - 1)), 1)
        G = jnp.zeros((HALF - 1, N_EXPERTS * (HALF - 1)), jnp.float32)
        for k in range(TOP_K):
            G = jnp.where(cc == idxs[k] * (HALF - 1) + rr, gs[k], G)
        mix = lax.dot_general(G, ctxc_ref[...], (((1,), (0,)), ((), ())),
                              preferred_element_type=jnp.float32)

        mid_ref[0:HALF, :] = ctxg_ref[...]
        mid_ref[HALF:N_CTX - 1, :] = mix
        mid_ref[N_CTX - 1:N_CTX, :] = ctx_s + ws_b_ref[...]

    out_ref[:, 0:1, :] = prefix_ref[...]
    out_ref[:, 1:N_CTX + 1, :] = jnp.broadcast_to(
        mid_ref[...][None], (BLK, N_CTX, CTX_DIM))
    out_ref[:, N_CTX + 1:, :] = suffix_ref[...]


def kernel(rad, shared, ctx_g, ctx_c, Ws_w, Ws_b, w_gate,
           token_prefix, token_suffix, tokenized_prompts):
    prompts, aux = pl.pallas_call(
        _fused_body,
        grid=(N_CLS // BLK,),
        in_specs=[
            pl.BlockSpec((1, 512), lambda i: (0, 0)),
            pl.BlockSpec((512, N_EXPERTS), lambda i: (0, 0)),
            pl.BlockSpec((1, 256), lambda i: (0, 0)),
            pl.BlockSpec((CTX_DIM, 256), lambda i: (0, 0)),
            pl.BlockSpec((1, CTX_DIM), lambda i: (0, 0)),
            pl.BlockSpec((HALF, CTX_DIM), lambda i: (0, 0)),
            pl.BlockSpec((N_EXPERTS * (HALF - 1), CTX_DIM), lambda i: (0, 0)),
            pl.BlockSpec((BLK, 1, CTX_DIM), lambda i: (i, 0, 0)),
            pl.BlockSpec((BLK, SUF_LEN, CTX_DIM), lambda i: (i, 0, 0)),
        ],
        out_specs=(
            pl.BlockSpec((BLK, SEQ_LEN, CTX_DIM), lambda i: (i, 0, 0)),
            pl.BlockSpec((1, 1), lambda i: (0, 0)),
        ),
        out_shape=(
            jax.ShapeDtypeStruct((N_CLS, SEQ_LEN, CTX_DIM), jnp.float32),
            jax.ShapeDtypeStruct((1, 1), jnp.float32),
        ),
        scratch_shapes=[pltpu.VMEM((N_CTX, CTX_DIM), jnp.float32)],
    )(rad, w_gate, shared, Ws_w, Ws_b.reshape(1, CTX_DIM), ctx_g, ctx_c,
      token_prefix, token_suffix)
    return prompts, tokenized_prompts, aux.reshape(())
